# Initial kernel scaffold; baseline (speedup 1.0000x reference)
import jax
import jax.numpy as jnp
from jax.experimental import pallas as pl


def kernel(z, codebook):
    raise NotImplementedError("write your pallas kernel here")


# speedup vs baseline: 1.0934x; 1.0934x over previous
import functools

import jax
import jax.numpy as jnp
from jax import lax
from jax.experimental import pallas as pl
from jax.experimental.pallas import tpu as pltpu
from jax.experimental.pallas import tpu_sc as plsc

N_CODES = 8192
DIM = 32
N_ROWS = 8192
KB = 512
RB = 1024

_NC = 2
_NS = 16
_NW = _NC * _NS
_BPW = N_ROWS // _NW
_CHUNK = 128


def _argmin_body(z_ref, cb_ref, out_ref, mn_ref, arg_ref):
    k = pl.program_id(1)
    zb = z_ref[0]
    cbt = cb_ref[...]
    zn = jnp.sum(zb * zb, axis=0, keepdims=True)
    cn = jnp.sum(cbt * cbt, axis=1, keepdims=True)
    mm = lax.dot_general(cbt, zb, (((1,), (0,)), ((), ())),
                         preferred_element_type=jnp.float32)
    d = (zn + cn) - 2.0 * mm
    tmin = jnp.min(d, axis=0, keepdims=True)
    ki = lax.broadcasted_iota(jnp.int32, d.shape, 0)
    targ = jnp.min(jnp.where(d == tmin, ki, jnp.int32(2 ** 30)),
                   axis=0, keepdims=True) + k * KB

    @pl.when(k == 0)
    def _():
        mn_ref[...] = tmin
        arg_ref[...] = targ

    @pl.when(k > 0)
    def _():
        upd = tmin < mn_ref[...]
        mn_ref[...] = jnp.where(upd, tmin, mn_ref[...])
        arg_ref[...] = jnp.where(upd, targ, arg_ref[...])

    @pl.when(k == pl.num_programs(1) - 1)
    def _():
        out_ref[0] = arg_ref[...]


def _argmin_call(z3, codebook, *, interpret=False):
    nb = z3.shape[0]
    return pl.pallas_call(
        _argmin_body,
        grid=(nb, N_CODES // KB),
        in_specs=[
            pl.BlockSpec((1, DIM, RB), lambda b, k: (b, 0, 0)),
            pl.BlockSpec((KB, DIM), lambda b, k: (k, 0)),
        ],
        out_specs=pl.BlockSpec((1, 1, RB), lambda b, k: (b, 0, 0)),
        out_shape=jax.ShapeDtypeStruct((nb, 1, RB), jnp.int32),
        scratch_shapes=[
            pltpu.VMEM((1, RB), jnp.float32),
            pltpu.VMEM((1, RB), jnp.int32),
        ],
        interpret=interpret,
    )(z3, codebook)


def _gather_body(cb_hbm, idx_hbm, zp_hbm, zq_hbm, part_hbm,
                 idx_a, idx_b, rows_v, zp_v, part_v, sem):
    wid = lax.axis_index("s") * _NC + lax.axis_index("c")
    base = wid * _BPW
    pltpu.sync_copy(idx_hbm.at[wid * 2], idx_a)
    pltpu.sync_copy(idx_hbm.at[wid * 2 + 1], idx_b)
    pltpu.sync_copy(zp_hbm.at[pl.ds(base, _BPW)], zp_v)
    c0 = pltpu.async_copy(cb_hbm.at[idx_a], rows_v.at[pl.ds(0, _CHUNK)], sem)
    c1 = pltpu.async_copy(cb_hbm.at[idx_b], rows_v.at[pl.ds(_CHUNK, _CHUNK)], sem)
    c0.wait()
    c1.wait()
    pltpu.sync_copy(rows_v, zq_hbm.at[pl.ds(base, _BPW)])

    def body(i, acc):
        for c in range(DIM // 16):
            dq = rows_v[i, pl.ds(c * 16, 16)]
            dz = zp_v[i, pl.ds(c * 16, 16)]
            df = dq - dz
            acc = acc + df * df
        return acc

    acc = lax.fori_loop(0, _BPW, body, jnp.zeros((16,), jnp.float32))
    part_v[...] = acc
    pltpu.sync_copy(part_v, part_hbm.at[wid])


@functools.cache
def _gather_call():
    return pl.kernel(
        _gather_body,
        mesh=plsc.VectorSubcoreMesh(core_axis_name="c", subcore_axis_name="s"),
        out_type=[
            jax.ShapeDtypeStruct((N_ROWS, DIM), jnp.float32),
            jax.ShapeDtypeStruct((_NW, 16), jnp.float32),
        ],
        scratch_types=[
            pltpu.VMEM((_CHUNK,), jnp.int32),
            pltpu.VMEM((_CHUNK,), jnp.int32),
            pltpu.VMEM((_BPW, DIM), jnp.float32),
            pltpu.VMEM((_BPW, DIM), jnp.float32),
            pltpu.VMEM((16,), jnp.float32),
            pltpu.SemaphoreType.DMA,
        ],
        compiler_params=pltpu.CompilerParams(use_tc_tiling_on_sc=False),
    )


def kernel(z, codebook):
    b, c, h, w = z.shape
    z3 = z.reshape(b, c, h * w)
    idx8 = _argmin_call(z3, codebook)
    zp_flat = z3.transpose(0, 2, 1).reshape(N_ROWS, DIM)
    idx2 = idx8.reshape(N_ROWS // _CHUNK, _CHUNK)
    zq_flat, part = _gather_call()(codebook, idx2, zp_flat)
    loss = 1.25 * (jnp.sum(part) / jnp.float32(N_ROWS * DIM))
    zq = zq_flat.reshape(b, h, w, c).transpose(0, 3, 1, 2)
    return (zq, loss, idx8.reshape(b, 1, h, w))


# speedup vs baseline: 1.2652x; 1.1570x over previous
import functools

import jax
import jax.numpy as jnp
from jax import lax
from jax.experimental import pallas as pl
from jax.experimental.pallas import tpu as pltpu
from jax.experimental.pallas import tpu_sc as plsc

N_CODES = 8192
DIM = 32
AUG = 40
N_ROWS = 8192
KB = 512
RB = 1024

_NC = 2
_NS = 16
_NW = _NC * _NS
_BPW = N_ROWS // _NW
_CHUNK = 128

_BIG = 3e38


def _argmin_body(za_ref, cb_ref, idx_ref, loss_ref, mn_ref, arg_ref, acc_ref):
    b = pl.program_id(0)
    k = pl.program_id(1)
    za = za_ref[0]

    @pl.when(k == 0)
    def _():
        mn_ref[...] = jnp.full((8, RB), _BIG, jnp.float32)
        arg_ref[...] = jnp.zeros((8, RB), jnp.float32)
        zsq = jnp.sum(za * za) - jnp.float32(3 * RB)
        acc_ref[0] = jnp.where(b == 0, zsq, acc_ref[0] + zsq)

    m = mn_ref[...]
    a = arg_ref[...]
    SUB = 128
    for c in range(KB // SUB):
        cbc = cb_ref[pl.ds(c * SUB, SUB), :]
        mmc = lax.dot_general(cbc, za, (((1,), (0,)), ((), ())),
                              preferred_element_type=jnp.float32)
        for r in range(SUB // 8):
            g = k * (KB // 8) + c * (SUB // 8) + r
            dr = lax.slice(mmc, (r * 8, 0), (r * 8 + 8, RB))
            pred = dr < m
            m = jnp.minimum(m, dr)
            a = jnp.where(pred, jnp.float32(g), a)
    mn_ref[...] = m
    arg_ref[...] = a

    @pl.when(k == pl.num_programs(1) - 1)
    def _():
        sio = lax.broadcasted_iota(jnp.int32, (8, RB), 0).astype(jnp.float32)
        idxv = a * 8.0 + sio
        tm = jnp.min(m, axis=0, keepdims=True)
        code = jnp.min(jnp.where(m == tm, idxv, jnp.float32(_BIG)),
                       axis=0, keepdims=True)
        idx_ref[0] = code.astype(jnp.int32)
        acc_ref[0] = acc_ref[0] + jnp.sum(tm)

        @pl.when(b == pl.num_programs(0) - 1)
        def _():
            loss_ref[0, 0] = acc_ref[0]


def _argmin_call(z_aug, cb_aug, *, interpret=False):
    nb = z_aug.shape[0]
    return pl.pallas_call(
        _argmin_body,
        grid=(nb, N_CODES // KB),
        in_specs=[
            pl.BlockSpec((1, AUG, RB), lambda b, k: (b, 0, 0)),
            pl.BlockSpec((KB, AUG), lambda b, k: (k, 0)),
        ],
        out_specs=[
            pl.BlockSpec((1, 1, RB), lambda b, k: (b, 0, 0)),
            pl.BlockSpec(memory_space=pltpu.SMEM),
        ],
        out_shape=[
            jax.ShapeDtypeStruct((nb, 1, RB), jnp.int32),
            jax.ShapeDtypeStruct((1, 1), jnp.float32),
        ],
        scratch_shapes=[
            pltpu.VMEM((8, RB), jnp.float32),
            pltpu.VMEM((8, RB), jnp.float32),
            pltpu.SMEM((1,), jnp.float32),
        ],
        interpret=interpret,
    )(z_aug, cb_aug)


def _augment(z3, codebook):
    nb = z3.shape[0]
    cn = jnp.sum(codebook * codebook, axis=1, keepdims=True)
    cn_hi = cn.astype(jnp.bfloat16).astype(jnp.float32)
    rem = cn - cn_hi
    cn_l1 = rem.astype(jnp.bfloat16).astype(jnp.float32)
    cn_l2 = rem - cn_l1
    zero_c = jnp.zeros((N_CODES, AUG - DIM - 3), jnp.float32)
    cb_aug = jnp.concatenate([-2.0 * codebook, cn_hi, cn_l1, cn_l2, zero_c],
                             axis=1)
    pad = jnp.concatenate([jnp.ones((nb, 3, RB), jnp.float32),
                           jnp.zeros((nb, AUG - DIM - 3, RB), jnp.float32)],
                          axis=1)
    z_aug = jnp.concatenate([z3, pad], axis=1)
    return z_aug, cb_aug


def _gather_body(cb_hbm, idx_hbm, zq_hbm, idx_a, idx_b, rows_v, sem):
    wid = lax.axis_index("s") * _NC + lax.axis_index("c")
    base = wid * _BPW
    pltpu.sync_copy(idx_hbm.at[wid * 2], idx_a)
    pltpu.sync_copy(idx_hbm.at[wid * 2 + 1], idx_b)
    c0 = pltpu.async_copy(cb_hbm.at[idx_a], rows_v.at[pl.ds(0, _CHUNK)], sem)
    c1 = pltpu.async_copy(cb_hbm.at[idx_b], rows_v.at[pl.ds(_CHUNK, _CHUNK)], sem)
    c0.wait()
    c1.wait()
    pltpu.sync_copy(rows_v, zq_hbm.at[pl.ds(base, _BPW)])


@functools.cache
def _gather_call():
    return pl.kernel(
        _gather_body,
        mesh=plsc.VectorSubcoreMesh(core_axis_name="c", subcore_axis_name="s"),
        out_type=jax.ShapeDtypeStruct((N_ROWS, DIM), jnp.float32),
        scratch_types=[
            pltpu.VMEM((_CHUNK,), jnp.int32),
            pltpu.VMEM((_CHUNK,), jnp.int32),
            pltpu.VMEM((_BPW, DIM), jnp.float32),
            pltpu.SemaphoreType.DMA,
        ],
        compiler_params=pltpu.CompilerParams(use_tc_tiling_on_sc=False),
    )


def kernel(z, codebook):
    b, c, h, w = z.shape
    z3 = z.reshape(b, c, h * w)
    z_aug, cb_aug = _augment(z3, codebook)
    idx8, loss_sum = _argmin_call(z_aug, cb_aug)
    idx2 = idx8.reshape(N_ROWS // _CHUNK, _CHUNK)
    zq_flat = _gather_call()(codebook, idx2)
    loss = 1.25 * loss_sum[0, 0] / jnp.float32(N_ROWS * DIM)
    zq = zq_flat.reshape(b, h, w, c).transpose(0, 3, 1, 2)
    return (zq, loss, idx8.reshape(b, 1, h, w))


# speedup vs baseline: 1.2777x; 1.0099x over previous
import functools

import jax
import jax.numpy as jnp
from jax import lax
from jax.experimental import pallas as pl
from jax.experimental.pallas import tpu as pltpu
from jax.experimental.pallas import tpu_sc as plsc

N_CODES = 8192
DIM = 32
AUG = 40
N_ROWS = 8192
KB = 512
RB = 1024

_NC = 2
_NS = 16
_NW = _NC * _NS
_BPW = N_ROWS // _NW
_CHUNK = 128

_BIG = 3e38


def _argmin_body(z_ref, cb_ref, idx_ref, loss_ref, mn_ref, arg_ref, acc_ref):
    b = pl.program_id(0)
    k = pl.program_id(1)
    zb = z_ref[0]
    cbt = cb_ref[...]

    @pl.when(k == 0)
    def _():
        mn_ref[...] = jnp.full((8, RB), _BIG, jnp.float32)
        arg_ref[...] = jnp.zeros((8, RB), jnp.float32)

    m = mn_ref[...]
    a = arg_ref[...]
    zn = jnp.sum(zb * zb, axis=0, keepdims=True)
    cn = jnp.sum(cbt * cbt, axis=1, keepdims=True)
    mm2 = lax.dot_general(cbt * -2.0, zb, (((1,), (0,)), ((), ())),
                          preferred_element_type=jnp.float32)
    d = (zn + cn) + mm2
    for r in range(KB // 8):
        g = k * (KB // 8) + r
        dr = lax.slice(d, (r * 8, 0), (r * 8 + 8, RB))
        pred = dr < m
        m = jnp.minimum(m, dr)
        a = jnp.where(pred, jnp.float32(g), a)
    mn_ref[...] = m
    arg_ref[...] = a

    @pl.when(k == pl.num_programs(1) - 1)
    def _():
        sio = lax.broadcasted_iota(jnp.int32, (8, RB), 0).astype(jnp.float32)
        idxv = a * 8.0 + sio
        tm = jnp.min(m, axis=0, keepdims=True)
        code = jnp.min(jnp.where(m == tm, idxv, jnp.float32(_BIG)),
                       axis=0, keepdims=True)
        idx_ref[0] = code.astype(jnp.int32)
        s = jnp.sum(tm)
        acc_ref[0] = jnp.where(b == 0, s, acc_ref[0] + s)

        @pl.when(b == pl.num_programs(0) - 1)
        def _():
            loss_ref[0, 0] = acc_ref[0]


def _argmin_call(z3, codebook, *, interpret=False):
    nb = z3.shape[0]
    return pl.pallas_call(
        _argmin_body,
        grid=(nb, N_CODES // KB),
        in_specs=[
            pl.BlockSpec((1, DIM, RB), lambda b, k: (b, 0, 0)),
            pl.BlockSpec((KB, DIM), lambda b, k: (k, 0)),
        ],
        out_specs=[
            pl.BlockSpec((1, 1, RB), lambda b, k: (b, 0, 0)),
            pl.BlockSpec(memory_space=pltpu.SMEM),
        ],
        out_shape=[
            jax.ShapeDtypeStruct((nb, 1, RB), jnp.int32),
            jax.ShapeDtypeStruct((1, 1), jnp.float32),
        ],
        scratch_shapes=[
            pltpu.VMEM((8, RB), jnp.float32),
            pltpu.VMEM((8, RB), jnp.float32),
            pltpu.SMEM((1,), jnp.float32),
        ],
        interpret=interpret,
    )(z3, codebook)


def _gather_body(cb_hbm, idx_hbm, zq_hbm, idx_a, idx_b, rows_v, sem):
    wid = lax.axis_index("s") * _NC + lax.axis_index("c")
    base = wid * _BPW
    pltpu.sync_copy(idx_hbm.at[wid * 2], idx_a)
    pltpu.sync_copy(idx_hbm.at[wid * 2 + 1], idx_b)
    c0 = pltpu.async_copy(cb_hbm.at[idx_a], rows_v.at[pl.ds(0, _CHUNK)], sem)
    c1 = pltpu.async_copy(cb_hbm.at[idx_b], rows_v.at[pl.ds(_CHUNK, _CHUNK)], sem)
    c0.wait()
    c1.wait()
    pltpu.sync_copy(rows_v, zq_hbm.at[pl.ds(base, _BPW)])


@functools.cache
def _gather_call():
    return pl.kernel(
        _gather_body,
        mesh=plsc.VectorSubcoreMesh(core_axis_name="c", subcore_axis_name="s"),
        out_type=jax.ShapeDtypeStruct((N_ROWS, DIM), jnp.float32),
        scratch_types=[
            pltpu.VMEM((_CHUNK,), jnp.int32),
            pltpu.VMEM((_CHUNK,), jnp.int32),
            pltpu.VMEM((_BPW, DIM), jnp.float32),
            pltpu.SemaphoreType.DMA,
        ],
        compiler_params=pltpu.CompilerParams(use_tc_tiling_on_sc=False),
    )


def kernel(z, codebook):
    b, c, h, w = z.shape
    z3 = z.reshape(b, c, h * w)
    idx8, loss_sum = _argmin_call(z3, codebook)
    idx2 = idx8.reshape(N_ROWS // _CHUNK, _CHUNK)
    zq_flat = _gather_call()(codebook, idx2)
    loss = 1.25 * loss_sum[0, 0] / jnp.float32(N_ROWS * DIM)
    zq = zq_flat.reshape(b, h, w, c).transpose(0, 3, 1, 2)
    return (zq, loss, idx8.reshape(b, 1, h, w))


# speedup vs baseline: 1.9588x; 1.5331x over previous
import functools

import jax
import jax.numpy as jnp
from jax import lax
from jax.experimental import pallas as pl
from jax.experimental.pallas import tpu as pltpu
from jax.experimental.pallas import tpu_sc as plsc

N_CODES = 8192
DIM = 32
AUG = 40
N_ROWS = 8192
KB = 8192
RB = 1024

_NC = 2
_NS = 16
_NW = _NC * _NS
_BPW = N_ROWS // _NW
_CHUNK = 128

_BIG = 3e38


def _argmin_body(z_ref, cb_ref, idx_ref, loss_ref, mn_ref, arg_ref, acc_ref):
    b = pl.program_id(0)
    k = pl.program_id(1)
    zb = z_ref[0]
    cbt = cb_ref[...]

    @pl.when(k == 0)
    def _():
        mn_ref[...] = jnp.full((8, RB), _BIG, jnp.float32)
        arg_ref[...] = jnp.zeros((8, RB), jnp.float32)

    m = mn_ref[...]
    a = arg_ref[...]
    zn = jnp.sum(zb * zb, axis=0, keepdims=True)
    cn = jnp.sum(cbt * cbt, axis=1, keepdims=True)
    mm2 = lax.dot_general(cbt * -2.0, zb, (((1,), (0,)), ((), ())),
                          preferred_element_type=jnp.float32)
    d = (zn + cn) + mm2
    for r in range(KB // 8):
        g = k * (KB // 8) + r
        dr = lax.slice(d, (r * 8, 0), (r * 8 + 8, RB))
        pred = dr < m
        m = jnp.minimum(m, dr)
        a = jnp.where(pred, jnp.float32(g), a)
    mn_ref[...] = m
    arg_ref[...] = a

    @pl.when(k == pl.num_programs(1) - 1)
    def _():
        sio = lax.broadcasted_iota(jnp.int32, (8, RB), 0).astype(jnp.float32)
        idxv = a * 8.0 + sio
        tm = jnp.min(m, axis=0, keepdims=True)
        code = jnp.min(jnp.where(m == tm, idxv, jnp.float32(_BIG)),
                       axis=0, keepdims=True)
        idx_ref[0] = code.astype(jnp.int32)
        s = jnp.sum(tm)
        acc_ref[0] = jnp.where(b == 0, s, acc_ref[0] + s)

        @pl.when(b == pl.num_programs(0) - 1)
        def _():
            loss_ref[0, 0] = acc_ref[0]


def _argmin_call(z3, codebook, *, interpret=False):
    nb = z3.shape[0]
    return pl.pallas_call(
        _argmin_body,
        grid=(nb, N_CODES // KB),
        in_specs=[
            pl.BlockSpec((1, DIM, RB), lambda b, k: (b, 0, 0)),
            pl.BlockSpec((KB, DIM), lambda b, k: (k, 0)),
        ],
        out_specs=[
            pl.BlockSpec((1, 1, RB), lambda b, k: (b, 0, 0)),
            pl.BlockSpec(memory_space=pltpu.SMEM),
        ],
        out_shape=[
            jax.ShapeDtypeStruct((nb, 1, RB), jnp.int32),
            jax.ShapeDtypeStruct((1, 1), jnp.float32),
        ],
        scratch_shapes=[
            pltpu.VMEM((8, RB), jnp.float32),
            pltpu.VMEM((8, RB), jnp.float32),
            pltpu.SMEM((1,), jnp.float32),
        ],
        interpret=interpret,
    )(z3, codebook)


def _gather_body(cb_hbm, idx_hbm, zq_hbm, idx_a, idx_b, rows_v, sem):
    wid = lax.axis_index("s") * _NC + lax.axis_index("c")
    base = wid * _BPW
    pltpu.sync_copy(idx_hbm.at[wid * 2], idx_a)
    pltpu.sync_copy(idx_hbm.at[wid * 2 + 1], idx_b)
    c0 = pltpu.async_copy(cb_hbm.at[idx_a], rows_v.at[pl.ds(0, _CHUNK)], sem)
    c1 = pltpu.async_copy(cb_hbm.at[idx_b], rows_v.at[pl.ds(_CHUNK, _CHUNK)], sem)
    c0.wait()
    c1.wait()
    pltpu.sync_copy(rows_v, zq_hbm.at[pl.ds(base, _BPW)])


@functools.cache
def _gather_call():
    return pl.kernel(
        _gather_body,
        mesh=plsc.VectorSubcoreMesh(core_axis_name="c", subcore_axis_name="s"),
        out_type=jax.ShapeDtypeStruct((N_ROWS, DIM), jnp.float32),
        scratch_types=[
            pltpu.VMEM((_CHUNK,), jnp.int32),
            pltpu.VMEM((_CHUNK,), jnp.int32),
            pltpu.VMEM((_BPW, DIM), jnp.float32),
            pltpu.SemaphoreType.DMA,
        ],
        compiler_params=pltpu.CompilerParams(use_tc_tiling_on_sc=False),
    )


def kernel(z, codebook):
    b, c, h, w = z.shape
    z3 = z.reshape(b, c, h * w)
    idx8, loss_sum = _argmin_call(z3, codebook)
    idx2 = idx8.reshape(N_ROWS // _CHUNK, _CHUNK)
    zq_flat = _gather_call()(codebook, idx2)
    loss = 1.25 * loss_sum[0, 0] / jnp.float32(N_ROWS * DIM)
    zq = zq_flat.reshape(b, h, w, c).transpose(0, 3, 1, 2)
    return (zq, loss, idx8.reshape(b, 1, h, w))
